# baseline (speedup 1.0000x reference)
import functools

import jax
import jax.numpy as jnp
from jax import lax
from jax.experimental import pallas as pl
from jax.experimental.pallas import tpu as pltpu
from jax.experimental.pallas import tpu_sc as plsc

VOCAB = 100000
EMBED = 32
BATCH = 16384
FIELDS = 26
FPAD = 28

HALVES = 4
BH = BATCH // HALVES
NW = 32
PER_W = BH // NW
NB = 64
NCHUNK = PER_W // NB

BB = 512
TC_GRID = BH // BB
ROWS7 = BH * FPAD * EMBED // 128


def _gather_body(table_hbm, idx_hbm, out_hbm, idx_v, rows_v, sem):
    cid = lax.axis_index("c")
    sid = lax.axis_index("s")
    wid = sid * 2 + cid
    base = wid * PER_W

    def chunk_step(c, _):
        b0 = pl.multiple_of(base + c * NB, NB)
        pltpu.sync_copy(idx_hbm.at[pl.ds(b0, NB)], idx_v)
        copies = []
        for b in range(NB):
            copies.append(
                pltpu.async_copy(
                    table_hbm.at[idx_v.at[b]],
                    rows_v.at[b, pl.ds(0, FIELDS)],
                    sem,
                )
            )
        for cp in copies:
            cp.wait()
        pltpu.sync_copy(rows_v, out_hbm.at[pl.ds(b0, NB)])
        return 0

    lax.fori_loop(0, NCHUNK, chunk_step, 0)


def _sc_gather(table, idx):
    kern = functools.partial(
        pl.kernel,
        out_type=jax.ShapeDtypeStruct((BH, FPAD, EMBED), jnp.float32),
        mesh=plsc.VectorSubcoreMesh(core_axis_name="c", subcore_axis_name="s"),
        scratch_types=[
            pltpu.VMEM((NB, FIELDS), jnp.int32),
            pltpu.VMEM((NB, FPAD, EMBED), jnp.float32),
            pltpu.SemaphoreType.DMA,
        ],
        compiler_params=pltpu.CompilerParams(use_tc_tiling_on_sc=False),
    )(_gather_body)
    return kern(table, idx)


VB = 4096


def _detile_body(tt_ref, out_ref):
    x = tt_ref[...]
    xt = x.T
    xt = xt + 0.0
    xt = xt.reshape(VB // 4, 4, EMBED)
    xt = xt + 0.0
    out_ref[...] = xt.reshape(VB // 4, 128)


def _tc_detile(table_t):
    grid = (VOCAB + VB - 1) // VB
    return pl.pallas_call(
        _detile_body,
        grid=(grid,),
        in_specs=[pl.BlockSpec((EMBED, VB), lambda i: (0, i))],
        out_specs=pl.BlockSpec((VB // 4, 128), lambda i: (i, 0)),
        out_shape=jax.ShapeDtypeStruct((VOCAB // 4, 128), jnp.float32),
    )(table_t)


def _mul_body(rows_ref, val_ref, out_ref):
    x = rows_ref[...]
    x = x.reshape(BB, 7, 128)
    parts = [x[:, r, :].T for r in range(7)]
    y = jnp.concatenate(parts, axis=0)
    y = y[: FIELDS * EMBED]
    y = y + 0.0
    y = y.reshape(FIELDS, EMBED, BB)
    v = val_ref[...]
    out_ref[...] = y * v[:, None, :]


def _tc_mul_t(rows2d, val_t, half, prev=None):
    blk0 = half * TC_GRID
    in_specs = [
        pl.BlockSpec((BB * FPAD * EMBED // 128, 128), lambda i: (i, 0)),
        pl.BlockSpec((FIELDS, BB), lambda i: (0, i + blk0)),
    ]
    args = [rows2d, val_t]
    aliases = {}
    if prev is not None:
        in_specs.append(pl.BlockSpec(memory_space=pl.ANY))
        args.append(prev)
        aliases = {2: 0}
    body = _mul_body if prev is None else (lambda r, v, p, o: _mul_body(r, v, o))
    return pl.pallas_call(
        body,
        grid=(TC_GRID,),
        in_specs=in_specs,
        out_specs=pl.BlockSpec((FIELDS, EMBED, BB), lambda i: (0, 0, i + blk0)),
        out_shape=jax.ShapeDtypeStruct((FIELDS, EMBED, BATCH), jnp.float32),
        input_output_aliases=aliases,
    )(*args)


def kernel(embed_index, embed_value, table):
    idx = embed_index.astype(jnp.int32)
    val_t = embed_value.T
    table_lin = _tc_detile(table.T).reshape(VOCAB, EMBED)
    out_t = None
    rows = [_sc_gather(table_lin, idx[h * BH:(h + 1) * BH]) for h in range(HALVES)]
    for h in range(HALVES):
        rows2d = rows[h].reshape(ROWS7, 128)
        out_t = _tc_mul_t(rows2d, val_t, h, out_t)
    return jnp.transpose(out_t, (2, 0, 1))


# speedup vs baseline: 1.0242x; 1.0242x over previous
import functools

import jax
import jax.numpy as jnp
from jax import lax
from jax.experimental import pallas as pl
from jax.experimental.pallas import tpu as pltpu
from jax.experimental.pallas import tpu_sc as plsc

VOCAB = 100000
EMBED = 32
BATCH = 16384
FIELDS = 26
FPAD = 28

HALVES = 4
BH = BATCH // HALVES
NW = 32
PER_W = BH // NW
NB = 64
NCHUNK = PER_W // NB

BB = 512
TC_GRID = BH // BB
ROWS7 = BH * FPAD * EMBED // 128


def _gather_body(table_hbm, idx_hbm, out_hbm, idx_v, rows_v, sem):
    cid = lax.axis_index("c")
    sid = lax.axis_index("s")
    wid = sid * 2 + cid
    base = wid * PER_W

    def chunk_step(c, _):
        b0 = pl.multiple_of(base + c * NB, NB)
        pltpu.sync_copy(idx_hbm.at[pl.ds(b0, NB)], idx_v)
        copies = []
        for b in range(NB):
            copies.append(
                pltpu.async_copy(
                    table_hbm.at[idx_v.at[b]],
                    rows_v.at[b, pl.ds(0, FIELDS)],
                    sem,
                )
            )
        for cp in copies:
            cp.wait()
        pltpu.sync_copy(rows_v, out_hbm.at[pl.ds(b0, NB)])
        return 0

    lax.fori_loop(0, NCHUNK, chunk_step, 0)


def _sc_gather(table, idx):
    kern = functools.partial(
        pl.kernel,
        out_type=jax.ShapeDtypeStruct((BH, FPAD, EMBED), jnp.float32),
        mesh=plsc.VectorSubcoreMesh(core_axis_name="c", subcore_axis_name="s"),
        scratch_types=[
            pltpu.VMEM((NB, FIELDS), jnp.int32),
            pltpu.VMEM((NB, FPAD, EMBED), jnp.float32),
            pltpu.SemaphoreType.DMA,
        ],
        compiler_params=pltpu.CompilerParams(use_tc_tiling_on_sc=False),
    )(_gather_body)
    return kern(table, idx)


VB = 8192


def _detile_body(tt_ref, out_ref):
    x = tt_ref[...]
    xt = x.T
    xt = xt + 0.0
    xt = xt.reshape(VB // 4, 4, EMBED)
    xt = xt + 0.0
    out_ref[...] = xt.reshape(VB // 4, 128)


def _tc_detile(table_t):
    grid = (VOCAB + VB - 1) // VB
    return pl.pallas_call(
        _detile_body,
        grid=(grid,),
        in_specs=[pl.BlockSpec((EMBED, VB), lambda i: (0, i))],
        out_specs=pl.BlockSpec((VB // 4, 128), lambda i: (i, 0)),
        out_shape=jax.ShapeDtypeStruct((VOCAB // 4, 128), jnp.float32),
    )(table_t)


def _mul_body(rows_ref, val_ref, out_ref):
    x = rows_ref[...]
    x = x.reshape(BB, 7, 128)
    parts = [x[:, r, :].T for r in range(7)]
    y = jnp.concatenate(parts, axis=0)
    y = y[: FIELDS * EMBED]
    y = y + 0.0
    y = y.reshape(FIELDS, EMBED, BB)
    v = val_ref[...]
    out_ref[...] = y * v[:, None, :]


def _tc_mul_t(rows2d, val_t, half, prev=None):
    blk0 = half * TC_GRID
    in_specs = [
        pl.BlockSpec((BB * FPAD * EMBED // 128, 128), lambda i: (i, 0)),
        pl.BlockSpec((FIELDS, BB), lambda i: (0, i + blk0)),
    ]
    args = [rows2d, val_t]
    aliases = {}
    if prev is not None:
        in_specs.append(pl.BlockSpec(memory_space=pl.ANY))
        args.append(prev)
        aliases = {2: 0}
    body = _mul_body if prev is None else (lambda r, v, p, o: _mul_body(r, v, o))
    return pl.pallas_call(
        body,
        grid=(TC_GRID,),
        in_specs=in_specs,
        out_specs=pl.BlockSpec((FIELDS, EMBED, BB), lambda i: (0, 0, i + blk0)),
        out_shape=jax.ShapeDtypeStruct((FIELDS, EMBED, BATCH), jnp.float32),
        input_output_aliases=aliases,
    )(*args)


def kernel(embed_index, embed_value, table):
    idx = embed_index.astype(jnp.int32)
    val_t = embed_value.T
    table_lin = _tc_detile(table.T).reshape(VOCAB, EMBED)
    out_t = None
    rows = [_sc_gather(table_lin, idx[h * BH:(h + 1) * BH]) for h in range(HALVES)]
    for h in range(HALVES):
        rows2d = rows[h].reshape(ROWS7, 128)
        out_t = _tc_mul_t(rows2d, val_t, h, out_t)
    return jnp.transpose(out_t, (2, 0, 1))
